# baseline (speedup 1.0000x reference)
import functools

import jax
import jax.numpy as jnp
from jax import lax
from jax.experimental import pallas as pl
from jax.experimental.pallas import tpu as pltpu
from jax.experimental.pallas import tpu_sc as plsc

N_TOK = 32768
N_BATCH = 16
IN_DIM = 4
HIDDEN = 64
FEAT_DIM = 16

MLP_GRID = 8
MLP_TILE = N_TOK // MLP_GRID


def _mlp_body(feats_ref, w1_ref, b1_ref, w2_ref, b2_ref, h_ref):
    x = feats_ref[...]
    h = jnp.maximum(
        lax.dot_general(x, w1_ref[...], (((1,), (0,)), ((), ())),
                        preferred_element_type=jnp.float32) + b1_ref[...],
        0.0)
    h_ref[...] = lax.dot_general(
        h, w2_ref[...], (((1,), (0,)), ((), ())),
        preferred_element_type=jnp.float32) + b2_ref[...]


def _mlp(feats, W1, b1r, W2, b2r):
    return pl.pallas_call(
        _mlp_body,
        grid=(MLP_GRID,),
        in_specs=[
            pl.BlockSpec((MLP_TILE, IN_DIM), lambda i: (i, 0)),
            pl.BlockSpec((IN_DIM, HIDDEN), lambda i: (0, 0)),
            pl.BlockSpec((1, HIDDEN), lambda i: (0, 0)),
            pl.BlockSpec((HIDDEN, FEAT_DIM), lambda i: (0, 0)),
            pl.BlockSpec((1, FEAT_DIM), lambda i: (0, 0)),
        ],
        out_specs=pl.BlockSpec((MLP_TILE, FEAT_DIM), lambda i: (i, 0)),
        out_shape=jax.ShapeDtypeStruct((N_TOK, FEAT_DIM), jnp.float32),
    )(feats, W1, b1r, W2, b2r)


NC = 2
NS = 16
NW = NC * NS
CNK = N_TOK // NW
L = 16


def _segpart_body(h_hbm, ids_hbm, pmax_hbm, pcnt_hbm, hv, idsv, cntv, outv):
    c = lax.axis_index("c")
    s = lax.axis_index("s")
    wid = s * NC + c
    base = wid * CNK

    pltpu.sync_copy(h_hbm.at[pl.ds(base * FEAT_DIM, CNK * FEAT_DIM)], hv)
    pltpu.sync_copy(ids_hbm.at[pl.ds(base, CNK)], idsv)

    cntv[...] = jnp.zeros((L,), jnp.float32)
    ones = jnp.ones((L,), jnp.float32)
    for g in range(CNK // L):
        idg = idsv[pl.ds(g * L, L)]
        plsc.addupdate_scatter(cntv, [idg], ones)
    cnt = cntv[...]
    pref = plsc.cumsum(cnt).astype(jnp.int32)

    lane = lax.iota(jnp.int32, L)
    neg_inf = jnp.full((L,), -jnp.inf, jnp.float32)

    lo = jnp.int32(0)
    for b in range(N_BATCH):
        hi = jnp.max(jnp.where(lane == b, pref, 0))

        def body4(i, accs):
            a0, a1, a2, a3 = accs
            t = (lo + i * 4) * FEAT_DIM
            a0 = jnp.maximum(a0, hv[pl.ds(t, L)])
            a1 = jnp.maximum(a1, hv[pl.ds(t + L, L)])
            a2 = jnp.maximum(a2, hv[pl.ds(t + 2 * L, L)])
            a3 = jnp.maximum(a3, hv[pl.ds(t + 3 * L, L)])
            return a0, a1, a2, a3

        n4 = (hi - lo) // 4
        a0, a1, a2, a3 = lax.fori_loop(
            0, n4, body4, (neg_inf, neg_inf, neg_inf, neg_inf))

        def body1(t, a):
            return jnp.maximum(a, hv[pl.ds(t * FEAT_DIM, L)])

        a0 = lax.fori_loop(lo + n4 * 4, hi, body1, a0)
        outv[pl.ds(b * L, L)] = jnp.maximum(
            jnp.maximum(a0, a1), jnp.maximum(a2, a3))
        lo = hi

    pltpu.sync_copy(outv, pmax_hbm.at[pl.ds(wid * N_BATCH * FEAT_DIM,
                                            N_BATCH * FEAT_DIM)])
    pltpu.sync_copy(cntv, pcnt_hbm.at[pl.ds(wid * N_BATCH, N_BATCH)])


_segpart = pl.kernel(
    _segpart_body,
    out_type=(
        jax.ShapeDtypeStruct((NW * N_BATCH * FEAT_DIM,), jnp.float32),
        jax.ShapeDtypeStruct((NW * N_BATCH,), jnp.float32),
    ),
    mesh=plsc.VectorSubcoreMesh(
        core_axis_name="c", subcore_axis_name="s",
        num_cores=NC, num_subcores=NS),
    compiler_params=pltpu.CompilerParams(needs_layout_passes=False),
    scratch_types=[
        pltpu.VMEM((CNK * FEAT_DIM,), jnp.float32),
        pltpu.VMEM((CNK,), jnp.int32),
        pltpu.VMEM((L,), jnp.float32),
        pltpu.VMEM((N_BATCH * FEAT_DIM,), jnp.float32),
    ],
)


def _finalize_body(pmax_ref, pcnt_ref, out_ref):
    m = jnp.full((N_BATCH, FEAT_DIM), -jnp.inf, jnp.float32)
    for s in range(NW):
        m = jnp.maximum(m, pmax_ref[s * N_BATCH:(s + 1) * N_BATCH, :])
    ones = jnp.ones((NW, 1), jnp.float32)
    cT = lax.dot_general(pcnt_ref[...], ones, (((0,), (0,)), ((), ())),
                         preferred_element_type=jnp.float32)
    padded = cT < jnp.max(cT)
    m = jnp.where(padded, jnp.maximum(m, 0.0), m)
    norm = jnp.sqrt(jnp.sum(m * m, axis=1, keepdims=True))
    out_ref[...] = m / jnp.maximum(norm, 1e-12)


def _finalize(pmax, pcnt):
    return pl.pallas_call(
        _finalize_body,
        out_shape=jax.ShapeDtypeStruct((N_BATCH, FEAT_DIM), jnp.float32),
    )(pmax, pcnt)


@jax.jit
def kernel(feats, batch_ids, W1, b1, W2, b2):
    h = _mlp(feats, W1, b1.reshape(1, HIDDEN), W2, b2.reshape(1, FEAT_DIM))
    pmax, pcnt = _segpart(h.reshape(-1), batch_ids)
    return _finalize(pmax.reshape(NW * N_BATCH, FEAT_DIM),
                     pcnt.reshape(NW, N_BATCH))


# speedup vs baseline: 1.5155x; 1.5155x over previous
import functools

import jax
import jax.numpy as jnp
from jax import lax
from jax.experimental import pallas as pl
from jax.experimental.pallas import tpu as pltpu
from jax.experimental.pallas import tpu_sc as plsc

N_TOK = 32768
N_BATCH = 16
IN_DIM = 4
HIDDEN = 64
FEAT_DIM = 16

MLP_GRID = 4
MLP_TILE = N_TOK // MLP_GRID


def _mlp_body(xt_ref, w1_ref, b1_ref, w2_ref, b2_ref, ht_ref):
    xt = xt_ref[...]
    h1t = lax.dot_general(w1_ref[...], xt, (((0,), (0,)), ((), ())),
                          preferred_element_type=jnp.float32)
    h1t = jnp.maximum(h1t + jnp.transpose(b1_ref[...]), 0.0)
    ht_ref[...] = lax.dot_general(
        w2_ref[...], h1t, (((0,), (0,)), ((), ())),
        preferred_element_type=jnp.float32) + jnp.transpose(b2_ref[...])


def _mlp(xt, W1, b1r, W2, b2r):
    return pl.pallas_call(
        _mlp_body,
        grid=(MLP_GRID,),
        in_specs=[
            pl.BlockSpec((IN_DIM, MLP_TILE), lambda i: (0, i)),
            pl.BlockSpec((IN_DIM, HIDDEN), lambda i: (0, 0)),
            pl.BlockSpec((1, HIDDEN), lambda i: (0, 0)),
            pl.BlockSpec((HIDDEN, FEAT_DIM), lambda i: (0, 0)),
            pl.BlockSpec((1, FEAT_DIM), lambda i: (0, 0)),
        ],
        out_specs=pl.BlockSpec((FEAT_DIM, MLP_TILE), lambda i: (0, i)),
        out_shape=jax.ShapeDtypeStruct((FEAT_DIM, N_TOK), jnp.float32),
    )(xt, W1, b1r, W2, b2r)


NC = 2
NS = 16
NW = NC * NS
CNK = N_TOK // NW
L = 16


def _segpart_body(h_hbm, ids_hbm, pmax_hbm, pcnt_hbm, hv, idsv, cntv, outv,
                  sem):
    c = lax.axis_index("c")
    s = lax.axis_index("s")
    wid = s * NC + c
    base = wid * CNK

    descs = [
        pltpu.async_copy(h_hbm.at[pl.ds(f * N_TOK + base, CNK)],
                         hv.at[pl.ds(f * CNK, CNK)], sem)
        for f in range(FEAT_DIM)
    ]
    pltpu.sync_copy(ids_hbm.at[pl.ds(base, CNK)], idsv)
    for d in descs:
        d.wait()

    cntv[...] = jnp.zeros((L,), jnp.float32)
    ones = jnp.ones((L,), jnp.float32)
    for g in range(CNK // L):
        idg = idsv[pl.ds(g * L, L)]
        plsc.addupdate_scatter(cntv, [idg], ones)
    cnt = cntv[...]
    pref = plsc.cumsum(cnt).astype(jnp.int32)

    lane = lax.iota(jnp.int32, L)
    colbase = lane * CNK
    neg_inf = jnp.full((L,), -jnp.inf, jnp.float32)

    lo = jnp.int32(0)
    for b in range(N_BATCH):
        hi = jnp.max(jnp.where(lane == b, pref, 0))

        def body4(i, accs):
            a0, a1, a2, a3 = accs
            idx = colbase + (lo + i * 4)
            a0 = jnp.maximum(a0, plsc.load_gather(hv, [idx]))
            a1 = jnp.maximum(a1, plsc.load_gather(hv, [idx + 1]))
            a2 = jnp.maximum(a2, plsc.load_gather(hv, [idx + 2]))
            a3 = jnp.maximum(a3, plsc.load_gather(hv, [idx + 3]))
            return a0, a1, a2, a3

        n4 = (hi - lo) // 4
        a0, a1, a2, a3 = lax.fori_loop(
            0, n4, body4, (neg_inf, neg_inf, neg_inf, neg_inf))

        def body1(t, a):
            return jnp.maximum(a, plsc.load_gather(hv, [colbase + t]))

        a0 = lax.fori_loop(lo + n4 * 4, hi, body1, a0)
        outv[pl.ds(b * L, L)] = jnp.maximum(
            jnp.maximum(a0, a1), jnp.maximum(a2, a3))
        lo = hi

    pltpu.sync_copy(outv, pmax_hbm.at[pl.ds(wid * N_BATCH * FEAT_DIM,
                                            N_BATCH * FEAT_DIM)])
    pltpu.sync_copy(cntv, pcnt_hbm.at[pl.ds(wid * N_BATCH, N_BATCH)])


_segpart = pl.kernel(
    _segpart_body,
    out_type=(
        jax.ShapeDtypeStruct((NW * N_BATCH * FEAT_DIM,), jnp.float32),
        jax.ShapeDtypeStruct((NW * N_BATCH,), jnp.float32),
    ),
    mesh=plsc.VectorSubcoreMesh(
        core_axis_name="c", subcore_axis_name="s",
        num_cores=NC, num_subcores=NS),
    compiler_params=pltpu.CompilerParams(needs_layout_passes=False),
    scratch_types=[
        pltpu.VMEM((CNK * FEAT_DIM,), jnp.float32),
        pltpu.VMEM((CNK,), jnp.int32),
        pltpu.VMEM((L,), jnp.float32),
        pltpu.VMEM((N_BATCH * FEAT_DIM,), jnp.float32),
        pltpu.SemaphoreType.DMA,
    ],
)


def _finalize_body(pmax_ref, pcnt_ref, out_ref):
    m = jnp.full((N_BATCH, FEAT_DIM), -jnp.inf, jnp.float32)
    for w in range(NW):
        m = jnp.maximum(m, pmax_ref[w * N_BATCH:(w + 1) * N_BATCH, :])
    cnt = pcnt_ref[...]
    ones = jnp.ones((NW, 1), jnp.float32)
    cT = lax.dot_general(cnt, ones, (((0,), (0,)), ((), ())),
                         preferred_element_type=jnp.float32)
    padded = cT < jnp.max(cT)
    m = jnp.where(padded, jnp.maximum(m, 0.0), m)
    norm = jnp.sqrt(jnp.sum(m * m, axis=1, keepdims=True))
    out_ref[...] = m / jnp.maximum(norm, 1e-12)


def _finalize(pmax, pcnt):
    return pl.pallas_call(
        _finalize_body,
        out_shape=jax.ShapeDtypeStruct((N_BATCH, FEAT_DIM), jnp.float32),
    )(pmax, pcnt)


@jax.jit
def kernel(feats, batch_ids, W1, b1, W2, b2):
    ht = _mlp(feats.T, W1, b1.reshape(1, HIDDEN), W2, b2.reshape(1, FEAT_DIM))
    pmax, pcnt = _segpart(ht.reshape(-1), batch_ids)
    return _finalize(pmax.reshape(NW * N_BATCH, FEAT_DIM),
                     pcnt.reshape(NW, N_BATCH))


# speedup vs baseline: 1.8214x; 1.2018x over previous
import functools

import jax
import jax.numpy as jnp
from jax import lax
from jax.experimental import pallas as pl
from jax.experimental.pallas import tpu as pltpu
from jax.experimental.pallas import tpu_sc as plsc

N_TOK = 32768
N_BATCH = 16
IN_DIM = 4
HIDDEN = 64
FEAT_DIM = 16

MLP_GRID = 4
MLP_TILE = N_TOK // MLP_GRID


def _mlp_body(xt_ref, w1_ref, b1_ref, w2t_ref, b2_ref, ht_ref):
    xt = xt_ref[...]
    h1t = lax.dot_general(w1_ref[...], xt, (((0,), (0,)), ((), ())),
                          preferred_element_type=jnp.float32)
    h1t = jnp.maximum(h1t + jnp.transpose(b1_ref[...]), 0.0)
    ht_ref[...] = lax.dot_general(
        w2t_ref[...], h1t, (((1,), (0,)), ((), ())),
        preferred_element_type=jnp.float32) + jnp.transpose(b2_ref[...])


def _mlp(xt, W1, b1r, W2, b2r):
    return pl.pallas_call(
        _mlp_body,
        grid=(MLP_GRID,),
        in_specs=[
            pl.BlockSpec((IN_DIM, MLP_TILE), lambda i: (0, i)),
            pl.BlockSpec((IN_DIM, HIDDEN), lambda i: (0, 0)),
            pl.BlockSpec((1, HIDDEN), lambda i: (0, 0)),
            pl.BlockSpec((FEAT_DIM, HIDDEN), lambda i: (0, 0)),
            pl.BlockSpec((1, FEAT_DIM), lambda i: (0, 0)),
        ],
        out_specs=pl.BlockSpec((FEAT_DIM, MLP_TILE), lambda i: (0, i)),
        out_shape=jax.ShapeDtypeStruct((FEAT_DIM, N_TOK), jnp.float32),
    )(xt, W1, b1r, W2, b2r)


NC = 2
NS = 16
NW = NC * NS
CNK = N_TOK // NW
L = 16


def _segpart_body(h_hbm, ids_hbm, pmax_hbm, pcnt_hbm, hv, idsv, cntv, outv,
                  sem):
    c = lax.axis_index("c")
    s = lax.axis_index("s")
    wid = s * NC + c
    base = wid * CNK

    descs = [
        pltpu.async_copy(h_hbm.at[pl.ds(f * N_TOK + base, CNK)],
                         hv.at[pl.ds(f * CNK, CNK)], sem)
        for f in range(FEAT_DIM)
    ]
    pltpu.sync_copy(ids_hbm.at[pl.ds(base, CNK)], idsv)
    for d in descs:
        d.wait()

    cntv[...] = jnp.zeros((L,), jnp.float32)
    ones = jnp.ones((L,), jnp.float32)
    for g in range(CNK // L):
        idg = idsv[pl.ds(g * L, L)]
        plsc.addupdate_scatter(cntv, [idg], ones)
    cnt = cntv[...]
    pref = plsc.cumsum(cnt).astype(jnp.int32)

    lane = lax.iota(jnp.int32, L)
    neg_inf = jnp.full((L,), -jnp.inf, jnp.float32)
    last_lane = lane == (L - 1)

    lo = jnp.int32(0)
    for b in range(N_BATCH):
        hi = jnp.max(jnp.where(lane == b, pref, 0))

        def gbody(g, accs):
            t0 = g * L
            pos = lane + t0
            m = (pos >= lo) & (pos < hi)
            return tuple(
                jnp.maximum(accs[f],
                            jnp.where(m, hv[pl.ds(f * CNK + t0, L)], neg_inf))
                for f in range(FEAT_DIM))

        accs = lax.fori_loop(lo // L, (hi + L - 1) // L, gbody,
                             (neg_inf,) * FEAT_DIM)

        for f in range(FEAT_DIM):
            cm = plsc.cummax(accs[f])
            plsc.store_scatter(outv,
                               [jnp.full((L,), b * L + f, jnp.int32)], cm,
                               mask=last_lane)
        lo = hi

    pltpu.sync_copy(outv, pmax_hbm.at[pl.ds(wid * N_BATCH * FEAT_DIM,
                                            N_BATCH * FEAT_DIM)])
    pltpu.sync_copy(cntv, pcnt_hbm.at[pl.ds(wid * N_BATCH, N_BATCH)])


_segpart = pl.kernel(
    _segpart_body,
    out_type=(
        jax.ShapeDtypeStruct((NW * N_BATCH * FEAT_DIM,), jnp.float32),
        jax.ShapeDtypeStruct((NW * N_BATCH,), jnp.float32),
    ),
    mesh=plsc.VectorSubcoreMesh(
        core_axis_name="c", subcore_axis_name="s",
        num_cores=NC, num_subcores=NS),
    compiler_params=pltpu.CompilerParams(needs_layout_passes=False),
    scratch_types=[
        pltpu.VMEM((CNK * FEAT_DIM,), jnp.float32),
        pltpu.VMEM((CNK,), jnp.int32),
        pltpu.VMEM((L,), jnp.float32),
        pltpu.VMEM((N_BATCH * FEAT_DIM,), jnp.float32),
        pltpu.SemaphoreType.DMA,
    ],
)


def _finalize_body(pmax_ref, pcnt_ref, out_ref):
    m = jnp.full((N_BATCH, FEAT_DIM), -jnp.inf, jnp.float32)
    for w in range(NW):
        m = jnp.maximum(m, pmax_ref[w * N_BATCH:(w + 1) * N_BATCH, :])
    cnt = pcnt_ref[...]
    ones = jnp.ones((NW, 1), jnp.float32)
    cT = lax.dot_general(cnt, ones, (((0,), (0,)), ((), ())),
                         preferred_element_type=jnp.float32)
    padded = cT < jnp.max(cT)
    m = jnp.where(padded, jnp.maximum(m, 0.0), m)
    norm = jnp.sqrt(jnp.sum(m * m, axis=1, keepdims=True))
    out_ref[...] = m / jnp.maximum(norm, 1e-12)


def _finalize(pmax, pcnt):
    return pl.pallas_call(
        _finalize_body,
        out_shape=jax.ShapeDtypeStruct((N_BATCH, FEAT_DIM), jnp.float32),
    )(pmax, pcnt)


@jax.jit
def kernel(feats, batch_ids, W1, b1, W2, b2):
    ht = _mlp(feats.T, W1, b1.reshape(1, HIDDEN), W2.T,
              b2.reshape(1, FEAT_DIM))
    pmax, pcnt = _segpart(ht.reshape(-1), batch_ids)
    return _finalize(pmax.reshape(NW * N_BATCH, FEAT_DIM),
                     pcnt.reshape(NW, N_BATCH))


# speedup vs baseline: 1.9124x; 1.0500x over previous
import functools

import jax
import jax.numpy as jnp
from jax import lax
from jax.experimental import pallas as pl
from jax.experimental.pallas import tpu as pltpu
from jax.experimental.pallas import tpu_sc as plsc

N_TOK = 32768
N_BATCH = 16
IN_DIM = 4
HIDDEN = 64
FEAT_DIM = 16

MLP_GRID = 4
MLP_TILE = N_TOK // MLP_GRID


def _mlp_body(xt_ref, w1_ref, b1_ref, w2t_ref, b2_ref, ht_ref):
    xt = xt_ref[...]
    h1t = lax.dot_general(w1_ref[...], xt, (((0,), (0,)), ((), ())),
                          preferred_element_type=jnp.float32)
    h1t = jnp.maximum(h1t + jnp.transpose(b1_ref[...]), 0.0)
    ht_ref[...] = lax.dot_general(
        w2t_ref[...], h1t, (((1,), (0,)), ((), ())),
        preferred_element_type=jnp.float32) + jnp.transpose(b2_ref[...])


def _mlp(xt, W1, b1r, W2, b2r):
    return pl.pallas_call(
        _mlp_body,
        grid=(MLP_GRID,),
        in_specs=[
            pl.BlockSpec((IN_DIM, MLP_TILE), lambda i: (0, i)),
            pl.BlockSpec((IN_DIM, HIDDEN), lambda i: (0, 0)),
            pl.BlockSpec((1, HIDDEN), lambda i: (0, 0)),
            pl.BlockSpec((FEAT_DIM, HIDDEN), lambda i: (0, 0)),
            pl.BlockSpec((1, FEAT_DIM), lambda i: (0, 0)),
        ],
        out_specs=pl.BlockSpec((FEAT_DIM, MLP_TILE), lambda i: (0, i)),
        out_shape=jax.ShapeDtypeStruct((FEAT_DIM, N_TOK), jnp.float32),
    )(xt, W1, b1r, W2, b2r)


NC = 2
NS = 16
NW = NC * NS
CNK = N_TOK // NW
L = 16


def _segpart_body(h_hbm, ids_hbm, pmax_hbm, pcnt_hbm, hv, idsv, cntv, outv,
                  sem):
    c = lax.axis_index("c")
    s = lax.axis_index("s")
    wid = s * NC + c
    base = wid * CNK

    descs = [
        pltpu.async_copy(
            h_hbm.at[pl.ds(half * (8 * N_TOK) + 8192 * wid, 8192)],
            hv.at[pl.ds(half * 8192, 8192)], sem)
        for half in range(2)
    ]
    pltpu.sync_copy(ids_hbm.at[pl.ds(base, CNK)], idsv)
    for d in descs:
        d.wait()

    cntv[...] = jnp.zeros((L,), jnp.float32)
    ones = jnp.ones((L,), jnp.float32)
    for g in range(CNK // L):
        idg = idsv[pl.ds(g * L, L)]
        plsc.addupdate_scatter(cntv, [idg], ones)
    cnt = cntv[...]
    pref = plsc.cumsum(cnt).astype(jnp.int32)

    lane = lax.iota(jnp.int32, L)
    neg_inf = jnp.full((L,), -jnp.inf, jnp.float32)
    last_lane = lane == (L - 1)
    f_off = [(f // 8) * 8192 + (f % 8) * 128 for f in range(FEAT_DIM)]

    for b in range(N_BATCH):
        outv[pl.ds(b * L, L)] = neg_inf

    lo = jnp.int32(0)
    for b in range(N_BATCH):
        hi = jnp.max(jnp.where(lane == b, pref, 0))

        @pl.when(hi > lo)
        def _seg(lo=lo, hi=hi, b=b):
            def gbody(g, accs):
                pos = lane + g * L
                m = (pos >= lo) & (pos < hi)
                col = (g // 8) * 1024 + (g % 8) * L
                return tuple(
                    jnp.maximum(
                        accs[f],
                        jnp.where(m, hv[pl.ds(f_off[f] + col, L)], neg_inf))
                    for f in range(FEAT_DIM))

            accs = lax.fori_loop(lo // L, (hi + L - 1) // L, gbody,
                                 (neg_inf,) * FEAT_DIM)

            for f in range(FEAT_DIM):
                cm = plsc.cummax(accs[f])
                plsc.store_scatter(outv,
                                   [jnp.full((L,), b * L + f, jnp.int32)],
                                   cm, mask=last_lane)

        lo = hi

    pltpu.sync_copy(outv, pmax_hbm.at[pl.ds(wid * N_BATCH * FEAT_DIM,
                                            N_BATCH * FEAT_DIM)])
    pltpu.sync_copy(cntv, pcnt_hbm.at[pl.ds(wid * N_BATCH, N_BATCH)])


_segpart = pl.kernel(
    _segpart_body,
    out_type=(
        jax.ShapeDtypeStruct((NW * N_BATCH * FEAT_DIM,), jnp.float32),
        jax.ShapeDtypeStruct((NW * N_BATCH,), jnp.float32),
    ),
    mesh=plsc.VectorSubcoreMesh(
        core_axis_name="c", subcore_axis_name="s",
        num_cores=NC, num_subcores=NS),
    compiler_params=pltpu.CompilerParams(needs_layout_passes=False),
    scratch_types=[
        pltpu.VMEM((CNK * FEAT_DIM,), jnp.float32),
        pltpu.VMEM((CNK,), jnp.int32),
        pltpu.VMEM((L,), jnp.float32),
        pltpu.VMEM((N_BATCH * FEAT_DIM,), jnp.float32),
        pltpu.SemaphoreType.DMA,
    ],
)


def _finalize_body(pmax_ref, pcnt_ref, out_ref):
    m = jnp.full((N_BATCH, FEAT_DIM), -jnp.inf, jnp.float32)
    for w in range(NW):
        m = jnp.maximum(m, pmax_ref[w * N_BATCH:(w + 1) * N_BATCH, :])
    cnt = pcnt_ref[...]
    ones = jnp.ones((NW, 1), jnp.float32)
    cT = lax.dot_general(cnt, ones, (((0,), (0,)), ((), ())),
                         preferred_element_type=jnp.float32)
    padded = cT < jnp.max(cT)
    m = jnp.where(padded, jnp.maximum(m, 0.0), m)
    norm = jnp.sqrt(jnp.sum(m * m, axis=1, keepdims=True))
    out_ref[...] = m / jnp.maximum(norm, 1e-12)


def _finalize(pmax, pcnt):
    return pl.pallas_call(
        _finalize_body,
        out_shape=jax.ShapeDtypeStruct((N_BATCH, FEAT_DIM), jnp.float32),
    )(pmax, pcnt)


@jax.jit
def kernel(feats, batch_ids, W1, b1, W2, b2):
    ht = _mlp(feats.T, W1, b1.reshape(1, HIDDEN), W2.T,
              b2.reshape(1, FEAT_DIM))
    hperm = ht.reshape(2, 8, 256, 128).transpose(0, 2, 1, 3).reshape(-1)
    pmax, pcnt = _segpart(hperm, batch_ids)
    return _finalize(pmax.reshape(NW * N_BATCH, FEAT_DIM),
                     pcnt.reshape(NW, N_BATCH))


# speedup vs baseline: 2.4638x; 1.2883x over previous
import functools

import jax
import jax.numpy as jnp
from jax import lax
from jax.experimental import pallas as pl
from jax.experimental.pallas import tpu as pltpu
from jax.experimental.pallas import tpu_sc as plsc

N_TOK = 32768
N_BATCH = 16
IN_DIM = 4
HIDDEN = 64
FEAT_DIM = 16

MLP_GRID = 2
MLP_TILE = N_TOK // MLP_GRID


def _mlp_body(xt_ref, w1_ref, b1_ref, w2t_ref, b2_ref, ht_ref):
    xt = xt_ref[...]
    h1t = lax.dot_general(w1_ref[...], xt, (((0,), (0,)), ((), ())),
                          preferred_element_type=jnp.float32)
    h1t = jnp.maximum(h1t + jnp.transpose(b1_ref[...]), 0.0)
    ht_ref[...] = lax.dot_general(
        w2t_ref[...], h1t, (((1,), (0,)), ((), ())),
        preferred_element_type=jnp.float32) + jnp.transpose(b2_ref[...])


def _mlp(xt, W1, b1r, W2t, b2r):
    return pl.pallas_call(
        _mlp_body,
        grid=(MLP_GRID,),
        in_specs=[
            pl.BlockSpec((IN_DIM, MLP_TILE), lambda i: (0, i)),
            pl.BlockSpec((IN_DIM, HIDDEN), lambda i: (0, 0)),
            pl.BlockSpec((1, HIDDEN), lambda i: (0, 0)),
            pl.BlockSpec((FEAT_DIM, HIDDEN), lambda i: (0, 0)),
            pl.BlockSpec((1, FEAT_DIM), lambda i: (0, 0)),
        ],
        out_specs=pl.BlockSpec((FEAT_DIM, MLP_TILE), lambda i: (0, i)),
        out_shape=jax.ShapeDtypeStruct((FEAT_DIM, N_TOK), jnp.float32),
    )(xt, W1, b1r, W2t, b2r)


NC = 2
NS = 16
NW = NC * NS
CNK = N_TOK // NW
L = 16
HALF_W = 8 * N_TOK


def _segpart_body(h_hbm, ids_hbm, pmax_hbm, pcnt_hbm, hv, idsv, cntv, outv,
                  sem):
    c = lax.axis_index("c")
    s = lax.axis_index("s")
    wid = s * NC + c
    base = wid * CNK

    descs = [
        pltpu.async_copy(
            h_hbm.at[pl.ds(half * HALF_W + 8192 * wid, 8192)],
            hv.at[pl.ds(half * 8192, 8192)], sem)
        for half in range(2)
    ]
    pltpu.sync_copy(ids_hbm.at[pl.ds(base, CNK)], idsv)
    for d in descs:
        d.wait()

    cntv[...] = jnp.zeros((L,), jnp.float32)
    ones = jnp.ones((L,), jnp.float32)

    def hbody(g, _):
        plsc.addupdate_scatter(cntv, [idsv[pl.ds(g * L, L)]], ones)
        return 0

    lax.fori_loop(0, CNK // L, hbody, 0)
    cnt = cntv[...]
    pref = plsc.cumsum(cnt).astype(jnp.int32)

    lane = lax.iota(jnp.int32, L)
    neg_inf = jnp.full((L,), -jnp.inf, jnp.float32)
    last_lane = lane == (L - 1)
    f_off = [(f // 8) * 8192 + (f % 8) * 128 for f in range(FEAT_DIM)]

    def init_body(b, _):
        outv[pl.ds(b * L, L)] = neg_inf
        return 0

    lax.fori_loop(0, N_BATCH, init_body, 0)

    def seg_body(b, lo):
        hi = jnp.max(jnp.where(lane == b, pref, 0))

        @pl.when(hi > lo)
        def _seg():
            def gbody(g, accs):
                pos = lane + g * L
                m = (pos >= lo) & (pos < hi)
                col = (g // 8) * 1024 + (g % 8) * L
                return tuple(
                    jnp.maximum(
                        accs[f],
                        jnp.where(m, hv[pl.ds(f_off[f] + col, L)], neg_inf))
                    for f in range(FEAT_DIM))

            accs = lax.fori_loop(lo // L, (hi + L - 1) // L, gbody,
                                 (neg_inf,) * FEAT_DIM)

            pos0 = (b // 8) * 128 + (b % 8) * L
            for f in range(FEAT_DIM):
                cm = plsc.cummax(accs[f])
                plsc.store_scatter(outv, [jnp.full((L,), pos0 + f,
                                                   jnp.int32)],
                                   cm, mask=last_lane)

        return hi

    lax.fori_loop(0, N_BATCH, seg_body, jnp.int32(0))

    pltpu.sync_copy(outv.at[pl.ds(0, 128)],
                    pmax_hbm.at[pl.ds(wid * 128, 128)])
    pltpu.sync_copy(outv.at[pl.ds(128, 128)],
                    pmax_hbm.at[pl.ds(NW * 128 + wid * 128, 128)])
    pltpu.sync_copy(cntv, pcnt_hbm.at[pl.ds(wid * N_BATCH, N_BATCH)])


_segpart = pl.kernel(
    _segpart_body,
    out_type=(
        jax.ShapeDtypeStruct((NW * N_BATCH * FEAT_DIM,), jnp.float32),
        jax.ShapeDtypeStruct((NW * N_BATCH,), jnp.float32),
    ),
    mesh=plsc.VectorSubcoreMesh(
        core_axis_name="c", subcore_axis_name="s",
        num_cores=NC, num_subcores=NS),
    compiler_params=pltpu.CompilerParams(needs_layout_passes=False),
    scratch_types=[
        pltpu.VMEM((CNK * FEAT_DIM,), jnp.float32),
        pltpu.VMEM((CNK,), jnp.int32),
        pltpu.VMEM((L,), jnp.float32),
        pltpu.VMEM((N_BATCH * FEAT_DIM,), jnp.float32),
        pltpu.SemaphoreType.DMA,
    ],
)


def _finalize_body(pmax_ref, pcnt_ref, out_ref):
    allm = pmax_ref[...]
    m0 = jnp.max(allm[:NW, :], axis=0, keepdims=True)
    m1 = jnp.max(allm[NW:, :], axis=0, keepdims=True)
    rows = [m0[:, (b % 8) * L:(b % 8) * L + L] for b in range(8)]
    rows += [m1[:, (b % 8) * L:(b % 8) * L + L] for b in range(8, 16)]
    m = jnp.concatenate(rows, axis=0)

    c = jnp.sum(pcnt_ref[...], axis=0, keepdims=True)
    c = c[:, :64] + c[:, 64:]
    c = c[:, :32] + c[:, 32:]
    c = c[:, :16] + c[:, 16:]
    padded = jnp.transpose(c) < jnp.max(c)
    m = jnp.where(padded, jnp.maximum(m, 0.0), m)
    norm = jnp.sqrt(jnp.sum(m * m, axis=1, keepdims=True))
    out_ref[...] = m / jnp.maximum(norm, 1e-12)


def _finalize(pmax, pcnt):
    return pl.pallas_call(
        _finalize_body,
        out_shape=jax.ShapeDtypeStruct((N_BATCH, FEAT_DIM), jnp.float32),
    )(pmax, pcnt)


@jax.jit
def kernel(feats, batch_ids, W1, b1, W2, b2):
    ht = _mlp(feats.T, W1, b1.reshape(1, HIDDEN), W2.T,
              b2.reshape(1, FEAT_DIM))
    hperm = ht.reshape(2, 8, 256, 128).transpose(0, 2, 1, 3).reshape(-1)
    pmax, pcnt = _segpart(hperm, batch_ids)
    return _finalize(pmax.reshape(2 * NW, 128), pcnt.reshape(NW // 8, 128))


# speedup vs baseline: 2.5146x; 1.0206x over previous
import functools

import jax
import jax.numpy as jnp
from jax import lax
from jax.experimental import pallas as pl
from jax.experimental.pallas import tpu as pltpu
from jax.experimental.pallas import tpu_sc as plsc

N_TOK = 32768
N_BATCH = 16
IN_DIM = 4
HIDDEN = 64
FEAT_DIM = 16

MLP_GRID = 1
MLP_TILE = N_TOK // MLP_GRID


def _mlp_body(xt_ref, w1_ref, b1_ref, w2t_ref, b2_ref, ht_ref):
    xt = xt_ref[...]
    h1t = lax.dot_general(w1_ref[...], xt, (((0,), (0,)), ((), ())),
                          preferred_element_type=jnp.float32)
    h1t = jnp.maximum(h1t + jnp.transpose(b1_ref[...]), 0.0)
    ht_ref[...] = lax.dot_general(
        w2t_ref[...], h1t, (((1,), (0,)), ((), ())),
        preferred_element_type=jnp.float32) + jnp.transpose(b2_ref[...])


def _mlp(xt, W1, b1r, W2t, b2r):
    return pl.pallas_call(
        _mlp_body,
        grid=(MLP_GRID,),
        in_specs=[
            pl.BlockSpec((IN_DIM, MLP_TILE), lambda i: (0, i)),
            pl.BlockSpec((IN_DIM, HIDDEN), lambda i: (0, 0)),
            pl.BlockSpec((1, HIDDEN), lambda i: (0, 0)),
            pl.BlockSpec((FEAT_DIM, HIDDEN), lambda i: (0, 0)),
            pl.BlockSpec((1, FEAT_DIM), lambda i: (0, 0)),
        ],
        out_specs=pl.BlockSpec((FEAT_DIM, MLP_TILE), lambda i: (0, i)),
        out_shape=jax.ShapeDtypeStruct((FEAT_DIM, N_TOK), jnp.float32),
    )(xt, W1, b1r, W2t, b2r)


NC = 2
NS = 16
NW = NC * NS
CNK = N_TOK // NW
L = 16
HALF_W = 8 * N_TOK


def _segpart_body(h_hbm, ids_hbm, pmax_hbm, pcnt_hbm, hv, idsv, cntv, outv,
                  sem):
    c = lax.axis_index("c")
    s = lax.axis_index("s")
    wid = s * NC + c
    base = wid * CNK

    descs = [
        pltpu.async_copy(
            h_hbm.at[pl.ds(half * HALF_W + 8192 * wid, 8192)],
            hv.at[pl.ds(half * 8192, 8192)], sem)
        for half in range(2)
    ]
    pltpu.sync_copy(ids_hbm.at[pl.ds(base, CNK)], idsv)

    cntv[...] = jnp.zeros((L,), jnp.float32)
    ones = jnp.ones((L,), jnp.float32)

    def hbody(g, _):
        plsc.addupdate_scatter(cntv, [idsv[pl.ds(g * L, L)]], ones)
        return 0

    lax.fori_loop(0, CNK // L, hbody, 0)
    cnt = cntv[...]
    pref = plsc.cumsum(cnt).astype(jnp.int32)

    for d in descs:
        d.wait()

    lane = lax.iota(jnp.int32, L)
    neg_inf = jnp.full((L,), -jnp.inf, jnp.float32)
    last_lane = lane == (L - 1)
    f_off = [(f // 8) * 8192 + (f % 8) * 128 for f in range(FEAT_DIM)]

    def init_body(b, _):
        outv[pl.ds(b * L, L)] = neg_inf
        return 0

    lax.fori_loop(0, N_BATCH, init_body, 0)

    def seg_body(b, lo):
        hi = jnp.max(jnp.where(lane == b, pref, 0))

        @pl.when(hi > lo)
        def _seg():
            def gbody(g, accs):
                pos = lane + g * L
                m = (pos >= lo) & (pos < hi)
                col = (g // 8) * 1024 + (g % 8) * L
                return tuple(
                    jnp.maximum(
                        accs[f],
                        jnp.where(m, hv[pl.ds(f_off[f] + col, L)], neg_inf))
                    for f in range(FEAT_DIM))

            accs = lax.fori_loop(lo // L, (hi + L - 1) // L, gbody,
                                 (neg_inf,) * FEAT_DIM)

            pos0 = (b // 8) * 128 + (b % 8) * L
            for f in range(FEAT_DIM):
                cm = plsc.cummax(accs[f])
                plsc.store_scatter(outv, [jnp.full((L,), pos0 + f,
                                                   jnp.int32)],
                                   cm, mask=last_lane)

        return hi

    lax.fori_loop(0, N_BATCH, seg_body, jnp.int32(0))

    pltpu.sync_copy(outv.at[pl.ds(0, 128)],
                    pmax_hbm.at[pl.ds(wid * 128, 128)])
    pltpu.sync_copy(outv.at[pl.ds(128, 128)],
                    pmax_hbm.at[pl.ds(NW * 128 + wid * 128, 128)])
    pltpu.sync_copy(cntv, pcnt_hbm.at[pl.ds(wid * N_BATCH, N_BATCH)])


_segpart = pl.kernel(
    _segpart_body,
    out_type=(
        jax.ShapeDtypeStruct((NW * N_BATCH * FEAT_DIM,), jnp.float32),
        jax.ShapeDtypeStruct((NW * N_BATCH,), jnp.float32),
    ),
    mesh=plsc.VectorSubcoreMesh(
        core_axis_name="c", subcore_axis_name="s",
        num_cores=NC, num_subcores=NS),
    compiler_params=pltpu.CompilerParams(needs_layout_passes=False),
    scratch_types=[
        pltpu.VMEM((CNK * FEAT_DIM,), jnp.float32),
        pltpu.VMEM((CNK,), jnp.int32),
        pltpu.VMEM((L,), jnp.float32),
        pltpu.VMEM((N_BATCH * FEAT_DIM,), jnp.float32),
        pltpu.SemaphoreType.DMA,
    ],
)


def _finalize_body(pmax_ref, pcnt_ref, out_ref):
    allm = pmax_ref[...]
    m0 = jnp.max(allm[:NW, :], axis=0, keepdims=True)
    m1 = jnp.max(allm[NW:, :], axis=0, keepdims=True)
    rows = [m0[:, (b % 8) * L:(b % 8) * L + L] for b in range(8)]
    rows += [m1[:, (b % 8) * L:(b % 8) * L + L] for b in range(8, 16)]
    m = jnp.concatenate(rows, axis=0)

    c = jnp.sum(pcnt_ref[...], axis=0, keepdims=True)
    c = c[:, :64] + c[:, 64:]
    c = c[:, :32] + c[:, 32:]
    c = c[:, :16] + c[:, 16:]
    padded = jnp.transpose(c) < jnp.max(c)
    m = jnp.where(padded, jnp.maximum(m, 0.0), m)
    norm = jnp.sqrt(jnp.sum(m * m, axis=1, keepdims=True))
    out_ref[...] = m / jnp.maximum(norm, 1e-12)


def _finalize(pmax, pcnt):
    return pl.pallas_call(
        _finalize_body,
        out_shape=jax.ShapeDtypeStruct((N_BATCH, FEAT_DIM), jnp.float32),
    )(pmax, pcnt)


@jax.jit
def kernel(feats, batch_ids, W1, b1, W2, b2):
    ht = _mlp(feats.T, W1, b1.reshape(1, HIDDEN), W2.T,
              b2.reshape(1, FEAT_DIM))
    hperm = ht.reshape(2, 8, 256, 128).transpose(0, 2, 1, 3).reshape(-1)
    pmax, pcnt = _segpart(hperm, batch_ids)
    return _finalize(pmax.reshape(2 * NW, 128), pcnt.reshape(NW // 8, 128))


# speedup vs baseline: 2.5187x; 1.0017x over previous
import functools

import jax
import jax.numpy as jnp
from jax import lax
from jax.experimental import pallas as pl
from jax.experimental.pallas import tpu as pltpu
from jax.experimental.pallas import tpu_sc as plsc

N_TOK = 32768
N_BATCH = 16
IN_DIM = 4
HIDDEN = 64
FEAT_DIM = 16

MLP_GRID = 2
MLP_TILE = N_TOK // MLP_GRID


def _mlp_body(xt_ref, w1_ref, b1_ref, w2t_ref, b2_ref, ht_ref):
    xt = xt_ref[...]
    h1t = lax.dot_general(w1_ref[...], xt, (((0,), (0,)), ((), ())),
                          preferred_element_type=jnp.float32)
    h1t = jnp.maximum(h1t + jnp.transpose(b1_ref[...]), 0.0)
    ht_ref[...] = lax.dot_general(
        w2t_ref[...], h1t, (((1,), (0,)), ((), ())),
        preferred_element_type=jnp.float32) + jnp.transpose(b2_ref[...])


def _mlp(xt, W1, b1r, W2t, b2r):
    return pl.pallas_call(
        _mlp_body,
        grid=(MLP_GRID,),
        in_specs=[
            pl.BlockSpec((IN_DIM, MLP_TILE), lambda i: (0, i)),
            pl.BlockSpec((IN_DIM, HIDDEN), lambda i: (0, 0)),
            pl.BlockSpec((1, HIDDEN), lambda i: (0, 0)),
            pl.BlockSpec((FEAT_DIM, HIDDEN), lambda i: (0, 0)),
            pl.BlockSpec((1, FEAT_DIM), lambda i: (0, 0)),
        ],
        out_specs=pl.BlockSpec((FEAT_DIM, MLP_TILE), lambda i: (0, i)),
        out_shape=jax.ShapeDtypeStruct((FEAT_DIM, N_TOK), jnp.float32),
    )(xt, W1, b1r, W2t, b2r)


NC = 2
NS = 16
NW = NC * NS
CNK = N_TOK // NW
L = 16
HALF_W = 8 * N_TOK


def _segpart_body(h_hbm, ids_hbm, pmax_hbm, pcnt_hbm, hv, idsv, cntv, outv,
                  sem):
    c = lax.axis_index("c")
    s = lax.axis_index("s")
    wid = s * NC + c
    base = wid * CNK

    descs = [
        pltpu.async_copy(
            h_hbm.at[pl.ds(half * HALF_W + 8192 * wid, 8192)],
            hv.at[pl.ds(half * 8192, 8192)], sem)
        for half in range(2)
    ]
    pltpu.sync_copy(ids_hbm.at[pl.ds(base, CNK)], idsv)

    cntv[...] = jnp.zeros((L,), jnp.float32)
    ones = jnp.ones((L,), jnp.float32)

    def hbody(g, _):
        plsc.addupdate_scatter(cntv, [idsv[pl.ds(g * L, L)]], ones)
        return 0

    lax.fori_loop(0, CNK // L, hbody, 0)
    cnt = cntv[...]
    pref = plsc.cumsum(cnt).astype(jnp.int32)

    for d in descs:
        d.wait()

    lane = lax.iota(jnp.int32, L)
    neg_inf = jnp.full((L,), -jnp.inf, jnp.float32)
    last_lane = lane == (L - 1)
    f_off = [(f // 8) * 8192 + (f % 8) * 128 for f in range(FEAT_DIM)]

    def init_body(b, _):
        outv[pl.ds(b * L, L)] = neg_inf
        return 0

    lax.fori_loop(0, N_BATCH, init_body, 0)

    def seg_body(b, lo):
        hi = jnp.max(jnp.where(lane == b, pref, 0))

        @pl.when(hi > lo)
        def _seg():
            def gbody(g, accs):
                pos = lane + g * L
                m = (pos >= lo) & (pos < hi)
                col = (g // 8) * 1024 + (g % 8) * L
                return tuple(
                    jnp.maximum(
                        accs[f],
                        jnp.where(m, hv[pl.ds(f_off[f] + col, L)], neg_inf))
                    for f in range(FEAT_DIM))

            accs = lax.fori_loop(lo // L, (hi + L - 1) // L, gbody,
                                 (neg_inf,) * FEAT_DIM)

            pos0 = (b // 8) * 128 + (b % 8) * L
            for f in range(FEAT_DIM):
                cm = plsc.cummax(accs[f])
                plsc.store_scatter(outv, [jnp.full((L,), pos0 + f,
                                                   jnp.int32)],
                                   cm, mask=last_lane)

        return hi

    lax.fori_loop(0, N_BATCH, seg_body, jnp.int32(0))

    pltpu.sync_copy(outv.at[pl.ds(0, 128)],
                    pmax_hbm.at[pl.ds(wid * 128, 128)])
    pltpu.sync_copy(outv.at[pl.ds(128, 128)],
                    pmax_hbm.at[pl.ds(NW * 128 + wid * 128, 128)])
    pltpu.sync_copy(cntv, pcnt_hbm.at[pl.ds(wid * N_BATCH, N_BATCH)])


_segpart = pl.kernel(
    _segpart_body,
    out_type=(
        jax.ShapeDtypeStruct((NW * N_BATCH * FEAT_DIM,), jnp.float32),
        jax.ShapeDtypeStruct((NW * N_BATCH,), jnp.float32),
    ),
    mesh=plsc.VectorSubcoreMesh(
        core_axis_name="c", subcore_axis_name="s",
        num_cores=NC, num_subcores=NS),
    compiler_params=pltpu.CompilerParams(needs_layout_passes=False),
    scratch_types=[
        pltpu.VMEM((CNK * FEAT_DIM,), jnp.float32),
        pltpu.VMEM((CNK,), jnp.int32),
        pltpu.VMEM((L,), jnp.float32),
        pltpu.VMEM((N_BATCH * FEAT_DIM,), jnp.float32),
        pltpu.SemaphoreType.DMA,
    ],
)


def _finalize_body(pmax_ref, pcnt_ref, out_ref):
    allm = pmax_ref[...]
    m0 = jnp.max(allm[:NW, :], axis=0, keepdims=True)
    m1 = jnp.max(allm[NW:, :], axis=0, keepdims=True)
    rows = [m0[:, (b % 8) * L:(b % 8) * L + L] for b in range(8)]
    rows += [m1[:, (b % 8) * L:(b % 8) * L + L] for b in range(8, 16)]
    m = jnp.concatenate(rows, axis=0)

    c = jnp.sum(pcnt_ref[...], axis=0, keepdims=True)
    c = c[:, :64] + c[:, 64:]
    c = c[:, :32] + c[:, 32:]
    c = c[:, :16] + c[:, 16:]
    padded = jnp.transpose(c) < jnp.max(c)
    m = jnp.where(padded, jnp.maximum(m, 0.0), m)
    norm = jnp.sqrt(jnp.sum(m * m, axis=1, keepdims=True))
    out_ref[...] = m / jnp.maximum(norm, 1e-12)


def _finalize(pmax, pcnt):
    return pl.pallas_call(
        _finalize_body,
        out_shape=jax.ShapeDtypeStruct((N_BATCH, FEAT_DIM), jnp.float32),
    )(pmax, pcnt)


@jax.jit
def kernel(feats, batch_ids, W1, b1, W2, b2):
    ht = _mlp(feats.T, W1, b1.reshape(1, HIDDEN), W2.T,
              b2.reshape(1, FEAT_DIM))
    hperm = ht.reshape(2, 8, 256, 128).transpose(0, 2, 1, 3).reshape(-1)
    pmax, pcnt = _segpart(hperm, batch_ids)
    return _finalize(pmax.reshape(2 * NW, 128), pcnt.reshape(NW // 8, 128))


# speedup vs baseline: 2.5246x; 1.0023x over previous
import functools

import jax
import jax.numpy as jnp
from jax import lax
from jax.experimental import pallas as pl
from jax.experimental.pallas import tpu as pltpu
from jax.experimental.pallas import tpu_sc as plsc

N_TOK = 32768
N_BATCH = 16
IN_DIM = 4
HIDDEN = 64
FEAT_DIM = 16

MLP_GRID = 2
MLP_TILE = N_TOK // MLP_GRID


def _mlp_body(xt_ref, w1_ref, b1_ref, w2t_ref, b2_ref, ht_ref):
    xt = xt_ref[...]
    h1t = lax.dot_general(w1_ref[...], xt, (((0,), (0,)), ((), ())),
                          preferred_element_type=jnp.float32)
    h1t = jnp.maximum(h1t + jnp.transpose(b1_ref[...]), 0.0)
    ht_ref[...] = lax.dot_general(
        w2t_ref[...], h1t, (((1,), (0,)), ((), ())),
        preferred_element_type=jnp.float32) + jnp.transpose(b2_ref[...])


def _mlp(xt, W1, b1r, W2t, b2r):
    return pl.pallas_call(
        _mlp_body,
        grid=(MLP_GRID,),
        in_specs=[
            pl.BlockSpec((IN_DIM, MLP_TILE), lambda i: (0, i)),
            pl.BlockSpec((IN_DIM, HIDDEN), lambda i: (0, 0)),
            pl.BlockSpec((1, HIDDEN), lambda i: (0, 0)),
            pl.BlockSpec((FEAT_DIM, HIDDEN), lambda i: (0, 0)),
            pl.BlockSpec((1, FEAT_DIM), lambda i: (0, 0)),
        ],
        out_specs=pl.BlockSpec((FEAT_DIM, MLP_TILE), lambda i: (0, i)),
        out_shape=jax.ShapeDtypeStruct((FEAT_DIM, N_TOK), jnp.float32),
    )(xt, W1, b1r, W2t, b2r)


NC = 2
NS = 16
NW = NC * NS
CNK = N_TOK // NW
L = 16
HALF_W = 8 * N_TOK


def _segpart_body(h_hbm, ids_hbm, pmax_hbm, pcnt_hbm, hv, idsv, cntv, outv,
                  sem):
    c = lax.axis_index("c")
    s = lax.axis_index("s")
    wid = s * NC + c
    base = wid * CNK

    descs = [
        pltpu.async_copy(
            h_hbm.at[pl.ds(half * HALF_W + 8192 * wid, 8192)],
            hv.at[pl.ds(half * 8192, 8192)], sem)
        for half in range(2)
    ]
    pltpu.sync_copy(ids_hbm.at[pl.ds(base, CNK)], idsv)

    cntv[...] = jnp.zeros((L,), jnp.float32)
    ones = jnp.ones((L,), jnp.float32)

    def hbody(g, _):
        plsc.addupdate_scatter(cntv, [idsv[pl.ds(g * L, L)]], ones)
        return 0

    lax.fori_loop(0, CNK // L, hbody, 0)
    cnt = cntv[...]
    pref = plsc.cumsum(cnt).astype(jnp.int32)

    for d in descs:
        d.wait()

    lane = lax.iota(jnp.int32, L)
    neg_inf = jnp.full((L,), -jnp.inf, jnp.float32)
    last_lane = lane == (L - 1)
    f_off = [(f // 8) * 8192 + (f % 8) * 128 for f in range(FEAT_DIM)]

    def init_body(b, _):
        outv[pl.ds(b * L, L)] = neg_inf
        return 0

    lax.fori_loop(0, N_BATCH, init_body, 0)

    def seg_body(b, lo):
        hi = jnp.max(jnp.where(lane == b, pref, 0))

        @pl.when(hi > lo)
        def _seg():
            glo = lo // L

            def gbody(i, accs):
                for j in range(2):
                    g = glo + i * 2 + j
                    pos = lane + g * L
                    m = (pos >= lo) & (pos < hi)
                    col = (g // 8) * 1024 + (g % 8) * L
                    accs = tuple(
                        jnp.maximum(
                            accs[f],
                            jnp.where(m, hv[pl.ds(f_off[f] + col, L)],
                                      neg_inf))
                        for f in range(FEAT_DIM))
                return accs

            ngrp = (hi + L - 1) // L - glo
            accs = lax.fori_loop(0, (ngrp + 1) // 2, gbody,
                                 (neg_inf,) * FEAT_DIM)

            pos0 = (b // 8) * 128 + (b % 8) * L
            for f in range(FEAT_DIM):
                cm = plsc.cummax(accs[f])
                plsc.store_scatter(outv, [jnp.full((L,), pos0 + f,
                                                   jnp.int32)],
                                   cm, mask=last_lane)

        return hi

    lax.fori_loop(0, N_BATCH, seg_body, jnp.int32(0))

    pltpu.sync_copy(outv.at[pl.ds(0, 128)],
                    pmax_hbm.at[pl.ds(wid * 128, 128)])
    pltpu.sync_copy(outv.at[pl.ds(128, 128)],
                    pmax_hbm.at[pl.ds(NW * 128 + wid * 128, 128)])
    pltpu.sync_copy(cntv, pcnt_hbm.at[pl.ds(wid * N_BATCH, N_BATCH)])


_segpart = pl.kernel(
    _segpart_body,
    out_type=(
        jax.ShapeDtypeStruct((NW * N_BATCH * FEAT_DIM,), jnp.float32),
        jax.ShapeDtypeStruct((NW * N_BATCH,), jnp.float32),
    ),
    mesh=plsc.VectorSubcoreMesh(
        core_axis_name="c", subcore_axis_name="s",
        num_cores=NC, num_subcores=NS),
    compiler_params=pltpu.CompilerParams(needs_layout_passes=False),
    scratch_types=[
        pltpu.VMEM((CNK * FEAT_DIM,), jnp.float32),
        pltpu.VMEM((CNK,), jnp.int32),
        pltpu.VMEM((L,), jnp.float32),
        pltpu.VMEM((N_BATCH * FEAT_DIM,), jnp.float32),
        pltpu.SemaphoreType.DMA,
    ],
)


def _finalize_body(pmax_ref, pcnt_ref, out_ref):
    allm = pmax_ref[...]
    m0 = jnp.max(allm[:NW, :], axis=0, keepdims=True)
    m1 = jnp.max(allm[NW:, :], axis=0, keepdims=True)
    rows = [m0[:, (b % 8) * L:(b % 8) * L + L] for b in range(8)]
    rows += [m1[:, (b % 8) * L:(b % 8) * L + L] for b in range(8, 16)]
    m = jnp.concatenate(rows, axis=0)

    c = jnp.sum(pcnt_ref[...], axis=0, keepdims=True)
    c = c[:, :64] + c[:, 64:]
    c = c[:, :32] + c[:, 32:]
    c = c[:, :16] + c[:, 16:]
    padded = jnp.transpose(c) < jnp.max(c)
    m = jnp.where(padded, jnp.maximum(m, 0.0), m)
    norm = jnp.sqrt(jnp.sum(m * m, axis=1, keepdims=True))
    out_ref[...] = m / jnp.maximum(norm, 1e-12)


def _finalize(pmax, pcnt):
    return pl.pallas_call(
        _finalize_body,
        out_shape=jax.ShapeDtypeStruct((N_BATCH, FEAT_DIM), jnp.float32),
    )(pmax, pcnt)


@jax.jit
def kernel(feats, batch_ids, W1, b1, W2, b2):
    ht = _mlp(feats.T, W1, b1.reshape(1, HIDDEN), W2.T,
              b2.reshape(1, FEAT_DIM))
    hperm = ht.reshape(2, 8, 256, 128).transpose(0, 2, 1, 3).reshape(-1)
    pmax, pcnt = _segpart(hperm, batch_ids)
    return _finalize(pmax.reshape(2 * NW, 128), pcnt.reshape(NW // 8, 128))


# speedup vs baseline: 2.5324x; 1.0031x over previous
import functools

import jax
import jax.numpy as jnp
from jax import lax
from jax.experimental import pallas as pl
from jax.experimental.pallas import tpu as pltpu
from jax.experimental.pallas import tpu_sc as plsc

N_TOK = 32768
N_BATCH = 16
IN_DIM = 4
HIDDEN = 64
FEAT_DIM = 16

MLP_GRID = 2
MLP_TILE = N_TOK // MLP_GRID


def _mlp_body(xt_ref, w1_ref, b1_ref, w2t_ref, b2_ref, ht_ref):
    xt = xt_ref[...]
    h1t = lax.dot_general(w1_ref[...], xt, (((0,), (0,)), ((), ())),
                          preferred_element_type=jnp.float32)
    h1t = jnp.maximum(h1t + jnp.transpose(b1_ref[...]), 0.0)
    ht_ref[...] = lax.dot_general(
        w2t_ref[...], h1t, (((1,), (0,)), ((), ())),
        preferred_element_type=jnp.float32) + jnp.transpose(b2_ref[...])


def _mlp(xt, W1, b1r, W2t, b2r):
    return pl.pallas_call(
        _mlp_body,
        grid=(MLP_GRID,),
        in_specs=[
            pl.BlockSpec((IN_DIM, MLP_TILE), lambda i: (0, i)),
            pl.BlockSpec((IN_DIM, HIDDEN), lambda i: (0, 0)),
            pl.BlockSpec((1, HIDDEN), lambda i: (0, 0)),
            pl.BlockSpec((FEAT_DIM, HIDDEN), lambda i: (0, 0)),
            pl.BlockSpec((1, FEAT_DIM), lambda i: (0, 0)),
        ],
        out_specs=pl.BlockSpec((FEAT_DIM, MLP_TILE), lambda i: (0, i)),
        out_shape=jax.ShapeDtypeStruct((FEAT_DIM, N_TOK), jnp.float32),
    )(xt, W1, b1r, W2t, b2r)


NC = 2
NS = 16
NW = NC * NS
CNK = N_TOK // NW
L = 16
HALF_W = 8 * N_TOK


def _segpart_body(h_hbm, ids_hbm, pmax_hbm, pcnt_hbm, hv, idsv, cntv, outv,
                  sem):
    c = lax.axis_index("c")
    s = lax.axis_index("s")
    wid = s * NC + c
    base = wid * CNK

    descs = [
        pltpu.async_copy(
            h_hbm.at[pl.ds(half * HALF_W + 8192 * wid, 8192)],
            hv.at[pl.ds(half * 8192, 8192)], sem)
        for half in range(2)
    ]
    pltpu.sync_copy(ids_hbm.at[pl.ds(base, CNK)], idsv)

    cntv[...] = jnp.zeros((L,), jnp.float32)
    ones = jnp.ones((L,), jnp.float32)

    def hbody(g, _):
        plsc.addupdate_scatter(cntv, [idsv[pl.ds(g * L, L)]], ones)
        return 0

    lax.fori_loop(0, CNK // L, hbody, 0)
    cnt = cntv[...]
    pref = plsc.cumsum(cnt).astype(jnp.int32)

    for d in descs:
        d.wait()

    lane = lax.iota(jnp.int32, L)
    neg_inf = jnp.full((L,), -jnp.inf, jnp.float32)
    last_lane = lane == (L - 1)
    f_off = [(f // 8) * 8192 + (f % 8) * 128 for f in range(FEAT_DIM)]

    def init_body(b, _):
        outv[pl.ds(b * L, L)] = neg_inf
        return 0

    lax.fori_loop(0, N_BATCH, init_body, 0)

    def seg_body(b, lo):
        hi = jnp.max(jnp.where(lane == b, pref, 0))

        @pl.when(hi > lo)
        def _seg():
            glo = lo // L

            def gbody(i, accs):
                for j in range(2):
                    g = glo + i * 2 + j
                    pos = lane + g * L
                    m = (pos >= lo) & (pos < hi)
                    col = (g // 8) * 1024 + (g % 8) * L
                    accs = tuple(
                        jnp.maximum(
                            accs[f],
                            jnp.where(m, hv[pl.ds(f_off[f] + col, L)],
                                      neg_inf))
                        for f in range(FEAT_DIM))
                return accs

            ngrp = (hi + L - 1) // L - glo
            accs = lax.fori_loop(0, (ngrp + 1) // 2, gbody,
                                 (neg_inf,) * FEAT_DIM)

            pos0 = (b // 8) * 128 + (b % 8) * L
            for f in range(FEAT_DIM):
                cm = plsc.cummax(accs[f])
                plsc.store_scatter(outv, [jnp.full((L,), pos0 + f,
                                                   jnp.int32)],
                                   cm, mask=last_lane)

        return hi

    lax.fori_loop(0, N_BATCH, seg_body, jnp.int32(0))

    d1 = pltpu.async_copy(outv.at[pl.ds(0, 128)],
                          pmax_hbm.at[pl.ds(wid * 128, 128)], sem)
    d2 = pltpu.async_copy(outv.at[pl.ds(128, 128)],
                          pmax_hbm.at[pl.ds(NW * 128 + wid * 128, 128)], sem)
    d3 = pltpu.async_copy(cntv, pcnt_hbm.at[pl.ds(wid * N_BATCH, N_BATCH)],
                          sem)
    d1.wait()
    d2.wait()
    d3.wait()


_segpart = pl.kernel(
    _segpart_body,
    out_type=(
        jax.ShapeDtypeStruct((NW * N_BATCH * FEAT_DIM,), jnp.float32),
        jax.ShapeDtypeStruct((NW * N_BATCH,), jnp.float32),
    ),
    mesh=plsc.VectorSubcoreMesh(
        core_axis_name="c", subcore_axis_name="s",
        num_cores=NC, num_subcores=NS),
    compiler_params=pltpu.CompilerParams(needs_layout_passes=False),
    scratch_types=[
        pltpu.VMEM((CNK * FEAT_DIM,), jnp.float32),
        pltpu.VMEM((CNK,), jnp.int32),
        pltpu.VMEM((L,), jnp.float32),
        pltpu.VMEM((N_BATCH * FEAT_DIM,), jnp.float32),
        pltpu.SemaphoreType.DMA,
    ],
)


def _finalize_body(pmax_ref, pcnt_ref, out_ref):
    allm = pmax_ref[...]
    m0 = jnp.max(allm[:NW, :], axis=0, keepdims=True)
    m1 = jnp.max(allm[NW:, :], axis=0, keepdims=True)
    rows = [m0[:, (b % 8) * L:(b % 8) * L + L] for b in range(8)]
    rows += [m1[:, (b % 8) * L:(b % 8) * L + L] for b in range(8, 16)]
    m = jnp.concatenate(rows, axis=0)

    c = jnp.sum(pcnt_ref[...], axis=0, keepdims=True)
    c = c[:, :64] + c[:, 64:]
    c = c[:, :32] + c[:, 32:]
    c = c[:, :16] + c[:, 16:]
    padded = jnp.transpose(c) < jnp.max(c)
    m = jnp.where(padded, jnp.maximum(m, 0.0), m)
    norm = jnp.sqrt(jnp.sum(m * m, axis=1, keepdims=True))
    out_ref[...] = m / jnp.maximum(norm, 1e-12)


def _finalize(pmax, pcnt):
    return pl.pallas_call(
        _finalize_body,
        out_shape=jax.ShapeDtypeStruct((N_BATCH, FEAT_DIM), jnp.float32),
    )(pmax, pcnt)


@jax.jit
def kernel(feats, batch_ids, W1, b1, W2, b2):
    ht = _mlp(feats.T, W1, b1.reshape(1, HIDDEN), W2.T,
              b2.reshape(1, FEAT_DIM))
    hperm = ht.reshape(2, 8, 256, 128).transpose(0, 2, 1, 3).reshape(-1)
    pmax, pcnt = _segpart(hperm, batch_ids)
    return _finalize(pmax.reshape(2 * NW, 128), pcnt.reshape(NW // 8, 128))
